# baseline (speedup 1.0000x reference)
import jax
import jax.numpy as jnp
from jax.experimental import pallas as pl
from jax.experimental.pallas import tpu as pltpu

_TNC = 256


def _mlp3_kernel(x_ref, w1_ref, b1_ref, w2_ref, b2_ref, w3_ref, b3_ref,
                 o_ref, xb, h1b, h2b):
    n = b1_ref.shape[1]
    nc = n // _TNC
    c = pl.program_id(1)

    @pl.when(c == 0)
    def _cast_x():
        xb[...] = x_ref[...].astype(jnp.bfloat16)

    @pl.when(c < nc)
    def _layer1_chunk():
        wc = w1_ref[...].astype(jnp.bfloat16)
        acc = jnp.dot(xb[...], wc, preferred_element_type=jnp.float32)
        off = c * _TNC
        h1b[:, pl.ds(off, _TNC)] = jnp.maximum(
            acc + b1_ref[:, pl.ds(off, _TNC)], 0.0).astype(jnp.bfloat16)

    @pl.when(c >= nc)
    def _layer2_chunk():
        wc = w2_ref[...].astype(jnp.bfloat16)
        acc = jnp.dot(h1b[...], wc, preferred_element_type=jnp.float32)
        off = (c - nc) * _TNC
        h2b[:, pl.ds(off, _TNC)] = jnp.maximum(
            acc + b2_ref[:, pl.ds(off, _TNC)], 0.0).astype(jnp.bfloat16)

    @pl.when(c == 2 * nc - 1)
    def _final():
        w3c = w3_ref[...].astype(jnp.bfloat16)
        y = jnp.dot(h2b[...], w3c, preferred_element_type=jnp.float32)
        o_ref[...] = y + b3_ref[...]


def _mlp3(x, w1, b1r, w2, b2r, w3, b3r, *, tm):
    M, K = x.shape
    N = w1.shape[1]
    O = w3.shape[1]
    nc = N // _TNC
    flops = 2 * M * K * N + 2 * M * N * N + 2 * M * N * O
    bytes_accessed = 4 * (M * K + K * N + N * N + N * O + M * O)

    return pl.pallas_call(
        _mlp3_kernel,
        out_shape=jax.ShapeDtypeStruct((M, O), jnp.float32),
        grid=(M // tm, 2 * nc),
        in_specs=[
            pl.BlockSpec((tm, K), lambda i, c: (i, 0)),
            pl.BlockSpec((K, _TNC), lambda i, c: (0, jnp.minimum(c, nc - 1))),
            pl.BlockSpec((1, N), lambda i, c: (0, 0)),
            pl.BlockSpec((K, _TNC), lambda i, c: (0, jnp.maximum(c - nc, 0))),
            pl.BlockSpec((1, N), lambda i, c: (0, 0)),
            pl.BlockSpec((N, O), lambda i, c: (0, 0)),
            pl.BlockSpec((1, O), lambda i, c: (0, 0)),
        ],
        out_specs=pl.BlockSpec((tm, O), lambda i, c: (i, 0)),
        scratch_shapes=[
            pltpu.VMEM((tm, K), jnp.bfloat16),
            pltpu.VMEM((tm, N), jnp.bfloat16),
            pltpu.VMEM((tm, N), jnp.bfloat16),
        ],
        compiler_params=pltpu.CompilerParams(
            dimension_semantics=("parallel", "arbitrary"),
        ),
        cost_estimate=pl.CostEstimate(
            flops=flops, transcendentals=0, bytes_accessed=bytes_accessed
        ),
    )(x, w1, b1r, w2, b2r, w3, b3r)


@jax.jit
def kernel(x, w1, b1, w2, b2, w3, b3):
    M = x.shape[0]
    N = w1.shape[1]
    O = w3.shape[1]
    tm = min(1024, max(M // 2, 8))
    return _mlp3(x, w1, b1.reshape(1, N), w2, b2.reshape(1, N),
                 w3, b3.reshape(1, O), tm=tm)


# speedup vs baseline: 1.0846x; 1.0846x over previous
import jax
import jax.numpy as jnp
from jax.experimental import pallas as pl
from jax.experimental.pallas import tpu as pltpu

def _mlp3_kernel(x_ref, w1_ref, b1_ref, w2_ref, b2_ref, w3_ref, b3_ref,
                 o_ref, xb, h1b, h2b):
    n = b1_ref.shape[1]
    tnc = w1_ref.shape[1]
    nc = n // tnc
    c = pl.program_id(1)

    @pl.when(c == 0)
    def _cast_x():
        xb[...] = x_ref[...].astype(jnp.bfloat16)

    @pl.when(c < nc)
    def _layer1_chunk():
        wc = w1_ref[...].astype(jnp.bfloat16)
        acc = jnp.dot(xb[...], wc, preferred_element_type=jnp.float32)
        off = c * tnc
        h1b[:, pl.ds(off, tnc)] = jnp.maximum(
            acc + b1_ref[:, pl.ds(off, tnc)], 0.0).astype(jnp.bfloat16)

    @pl.when(c >= nc)
    def _layer2_chunk():
        wc = w2_ref[...].astype(jnp.bfloat16)
        acc = jnp.dot(h1b[...], wc, preferred_element_type=jnp.float32)
        off = (c - nc) * tnc
        h2b[:, pl.ds(off, tnc)] = jnp.maximum(
            acc + b2_ref[:, pl.ds(off, tnc)], 0.0).astype(jnp.bfloat16)

    @pl.when(c == 2 * nc - 1)
    def _final():
        w3c = w3_ref[...].astype(jnp.bfloat16)
        y = jnp.dot(h2b[...], w3c, preferred_element_type=jnp.float32)
        o_ref[...] = y + b3_ref[...]


def _mlp3(x, w1, b1r, w2, b2r, w3, b3r, *, tm, tnc):
    M, K = x.shape
    N = w1.shape[1]
    O = w3.shape[1]
    nc = N // tnc
    flops = 2 * M * K * N + 2 * M * N * N + 2 * M * N * O
    bytes_accessed = 4 * (M * K + K * N + N * N + N * O + M * O)

    return pl.pallas_call(
        _mlp3_kernel,
        out_shape=jax.ShapeDtypeStruct((M, O), jnp.float32),
        grid=(M // tm, 2 * nc),
        in_specs=[
            pl.BlockSpec((tm, K), lambda i, c: (i, 0)),
            pl.BlockSpec((K, tnc), lambda i, c: (0, jnp.minimum(c, nc - 1))),
            pl.BlockSpec((1, N), lambda i, c: (0, 0)),
            pl.BlockSpec((K, tnc), lambda i, c: (0, jnp.maximum(c - nc, 0))),
            pl.BlockSpec((1, N), lambda i, c: (0, 0)),
            pl.BlockSpec((N, O), lambda i, c: (0, 0)),
            pl.BlockSpec((1, O), lambda i, c: (0, 0)),
        ],
        out_specs=pl.BlockSpec((tm, O), lambda i, c: (i, 0)),
        scratch_shapes=[
            pltpu.VMEM((tm, K), jnp.bfloat16),
            pltpu.VMEM((tm, N), jnp.bfloat16),
            pltpu.VMEM((tm, N), jnp.bfloat16),
        ],
        compiler_params=pltpu.CompilerParams(
            dimension_semantics=("parallel", "arbitrary"),
        ),
        cost_estimate=pl.CostEstimate(
            flops=flops, transcendentals=0, bytes_accessed=bytes_accessed
        ),
    )(x, w1, b1r, w2, b2r, w3, b3r)


@jax.jit
def kernel(x, w1, b1, w2, b2, w3, b3):
    M = x.shape[0]
    N = w1.shape[1]
    O = w3.shape[1]
    tm = min(1024, max(M // 2, 8))
    tnc = min(512, N // 2)
    return _mlp3(x, w1, b1.reshape(1, N), w2, b2.reshape(1, N),
                 w3, b3.reshape(1, O), tm=tm, tnc=tnc)
